# Initial kernel scaffold; baseline (speedup 1.0000x reference)
import jax
import jax.numpy as jnp
from jax.experimental import pallas as pl


def kernel(input_ids, emb_table):
    raise NotImplementedError("write your pallas kernel here")


# speedup vs baseline: 1.6366x; 1.6366x over previous
import functools

import jax
import jax.numpy as jnp
from jax import lax
from jax.experimental import pallas as pl
from jax.experimental.pallas import tpu as pltpu
from jax.experimental.pallas import tpu_sc as plsc

SEMANTIC = 4098
ACOUSTIC = 2050
NCB = 9
SEQ = 2048
D = 1024
NW = 32
LANES = 16
CHUNK = 64


def _sc_body(b_per_w, ids_hbm, table_hbm, out_hbm, idx_v, rows_v, sem):
    wid = lax.axis_index("s") * 2 + lax.axis_index("c")
    base = wid * b_per_w
    pltpu.sync_copy(ids_hbm.at[pl.ds(base, b_per_w)], idx_v)

    lane = lax.iota(jnp.int32, LANES)

    def add_off(i, carry):
        p = base + i * LANES + lane
        c = (p >> 11) % NCB
        off = jnp.where(c == 0, 0, SEMANTIC + (c - 1) * ACOUSTIC)
        idx_v[pl.ds(i * LANES, LANES)] = idx_v[pl.ds(i * LANES, LANES)] + off
        return carry

    lax.fori_loop(0, b_per_w // LANES, add_off, 0)

    def chunk_body(ci, carry):
        idx_c = idx_v.at[pl.ds(ci * CHUNK, CHUNK)]
        pltpu.async_copy(table_hbm.at[idx_c], rows_v, sem).wait()
        pltpu.sync_copy(rows_v, out_hbm.at[pl.ds(base + ci * CHUNK, CHUNK)])
        return carry

    lax.fori_loop(0, b_per_w // CHUNK, chunk_body, 0)


def _make_gather(b_total):
    b_per_w = b_total // NW
    mesh = plsc.VectorSubcoreMesh(core_axis_name="c", subcore_axis_name="s")
    return functools.partial(
        pl.kernel,
        mesh=mesh,
        out_type=jax.ShapeDtypeStruct((b_total, D), jnp.float32),
        scratch_types=[
            pltpu.VMEM((b_per_w,), jnp.int32),
            pltpu.VMEM((CHUNK, D), jnp.float32),
            pltpu.SemaphoreType.DMA,
        ],
    )(functools.partial(_sc_body, b_per_w))


def kernel(input_ids, emb_table):
    shape = input_ids.shape
    ids = input_ids.reshape(-1).astype(jnp.int32)
    out = _make_gather(ids.shape[0])(ids, emb_table)
    return out.reshape(shape + (D,))


# speedup vs baseline: 1.7204x; 1.0512x over previous
import functools

import jax
import jax.numpy as jnp
from jax import lax
from jax.experimental import pallas as pl
from jax.experimental.pallas import tpu as pltpu
from jax.experimental.pallas import tpu_sc as plsc

SEMANTIC = 4098
ACOUSTIC = 2050
NCB = 9
SEQ = 2048
D = 1024
NW = 32
LANES = 16
CHUNK = 48


def _sc_body(b_per_w, ids_hbm, table_hbm, out_hbm, idx_v,
             rows0, rows1, gsem0, gsem1, ssem0, ssem1):
    wid = lax.axis_index("s") * 2 + lax.axis_index("c")
    base = wid * b_per_w
    pltpu.sync_copy(ids_hbm.at[pl.ds(base, b_per_w)], idx_v)

    lane = lax.iota(jnp.int32, LANES)

    def add_off(i, carry):
        p = base + i * LANES + lane
        c = (p >> 11) % NCB
        off = jnp.where(c == 0, 0, SEMANTIC + (c - 1) * ACOUSTIC)
        idx_v[pl.ds(i * LANES, LANES)] = idx_v[pl.ds(i * LANES, LANES)] + off
        return carry

    lax.fori_loop(0, b_per_w // LANES, add_off, 0)

    nchunk = b_per_w // CHUNK
    rows = (rows0, rows1)
    gsem = (gsem0, gsem1)
    ssem = (ssem0, ssem1)

    def start_gather(ci):
        idx_c = idx_v.at[pl.ds(ci * CHUNK, CHUNK)]
        return pltpu.async_copy(table_hbm.at[idx_c], rows[ci % 2], gsem[ci % 2])

    gh = {0: start_gather(0)}
    sh = {}
    for ci in range(nchunk):
        if ci + 1 < nchunk:
            if ci >= 1:
                sh[ci - 1].wait()
            gh[ci + 1] = start_gather(ci + 1)
        gh[ci].wait()
        sh[ci] = pltpu.async_copy(
            rows[ci % 2], out_hbm.at[pl.ds(base + ci * CHUNK, CHUNK)],
            ssem[ci % 2])
    sh[nchunk - 2].wait()
    sh[nchunk - 1].wait()


def _make_gather(b_total):
    b_per_w = b_total // NW
    mesh = plsc.VectorSubcoreMesh(core_axis_name="c", subcore_axis_name="s")
    return functools.partial(
        pl.kernel,
        mesh=mesh,
        out_type=jax.ShapeDtypeStruct((b_total, D), jnp.float32),
        scratch_types=[
            pltpu.VMEM((b_per_w,), jnp.int32),
            pltpu.VMEM((CHUNK, D), jnp.float32),
            pltpu.VMEM((CHUNK, D), jnp.float32),
            pltpu.SemaphoreType.DMA,
            pltpu.SemaphoreType.DMA,
            pltpu.SemaphoreType.DMA,
            pltpu.SemaphoreType.DMA,
        ],
    )(functools.partial(_sc_body, b_per_w))


def kernel(input_ids, emb_table):
    shape = input_ids.shape
    ids = input_ids.reshape(-1).astype(jnp.int32)
    out = _make_gather(ids.shape[0])(ids, emb_table)
    return out.reshape(shape + (D,))
